# baseline (speedup 1.0000x reference)
import jax
import jax.numpy as jnp
from jax import lax
from jax.experimental import pallas as pl
from jax.experimental.pallas import tpu as pltpu
from jax.experimental.pallas import tpu_sc as plsc

M_ITEM = 1_000_000
DIM = 16
N_ROWS = 819_200
LR = 0.01

LANES = 16
C = 4_096
NFULL = M_ITEM // C
TAIL = M_ITEM - NFULL * C
NW = 32
B = 512
NEDGE = NFULL + 2


def _bval(bounds2, u):
    lane = lax.broadcasted_iota(jnp.int32, (LANES,), 0)
    row = bounds2[u // LANES]
    return jnp.sum(jnp.where(lane == (u % LANES), row, 0))


HS = N_ROWS // 16
HB = 2048


def _body(emb_t, items, grads4, out_t, bounds2, chunk, kv, gblk,
          hist, hkv, comb, shist):
    c = lax.axis_index("c")
    sid = lax.axis_index("s")
    w = sid * 2 + c

    zeros16 = jnp.zeros((LANES,), jnp.int32)
    ones16 = jnp.ones((LANES,), jnp.int32)
    for i in range(16):
        hist[pl.ds(i * LANES, LANES)] = zeros16

    def hchunk(h, carry):
        base = pl.multiple_of(sid * HS + h * HB, 128)
        pltpu.sync_copy(items.at[pl.ds(base, HB)], hkv)
        for q in range(HB // LANES):
            keys = hkv[pl.ds(q * LANES, LANES)]
            u = lax.shift_right_logical(keys, 12)
            plsc.addupdate_scatter(hist, [u], ones16)
        return carry
    lax.fori_loop(0, HS // HB, hchunk, 0)

    pltpu.sync_copy(hist, shist.at[sid])
    plsc.subcore_barrier()
    pltpu.sync_copy(shist, comb)

    lane = lax.broadcasted_iota(jnp.int32, (LANES,), 0)
    carry0 = jnp.zeros((), jnp.int32)
    for i in range(16):
        seg = comb[0, pl.ds(i * LANES, LANES)]
        for t in range(1, 16):
            seg = seg + comb[t, pl.ds(i * LANES, LANES)]
        incl = plsc.cumsum(seg)
        bounds2[i] = (incl - seg) + carry0
        carry0 = carry0 + jnp.sum(jnp.where(lane == 15, incl, 0))

    def do_unit(u, i0, width, lo, hi):
        pltpu.sync_copy(emb_t.at[:, pl.ds(i0, width)],
                        chunk.at[:, pl.ds(0, width)])

        lo_a = (lo // 128) * 128
        nblk = (hi - lo_a + (B - 1)) // B

        def blk(b, carry):
            s_nom = lo_a + b * B
            s = pl.multiple_of(jnp.minimum(s_nom, N_ROWS - B), 128)
            pltpu.sync_copy(items.at[pl.ds(s, B)], kv)
            pltpu.sync_copy(grads4.at[:, pl.ds(s // 128, B // 128)], gblk)
            glo = jnp.maximum(lo, s_nom)
            for q in range(B // LANES):
                keys = kv[pl.ds(q * LANES, LANES)]
                g = s + q * LANES + lax.broadcasted_iota(jnp.int32, (LANES,), 0)
                valid = (g >= glo) & (g < hi)
                idx = jnp.where(valid, keys - i0, 0)
                for d in range(DIM):
                    v = gblk[d // 8, q // 8, d % 8,
                             pl.ds((q % 8) * LANES, LANES)] * (-LR)
                    plsc.addupdate_scatter(chunk.at[d], [idx], v, mask=valid)
            return carry
        lax.fori_loop(0, nblk, blk, 0)

        pltpu.sync_copy(chunk.at[:, pl.ds(0, width)],
                        out_t.at[:, pl.ds(i0, width)])

    bounds2v = bounds2

    my_units = (NFULL - w + (NW - 1)) // NW

    def unit_body(k, carry):
        u = w + k * NW
        i0 = pl.multiple_of(u * C, 128)
        lo = _bval(bounds2v, u)
        hi = _bval(bounds2v, u + 1)
        do_unit(u, i0, C, lo, hi)
        return carry
    lax.fori_loop(0, my_units, unit_body, 0)

    @pl.when(w == NW - 1)
    def _():
        lo = _bval(bounds2v, NFULL)
        hi = _bval(bounds2v, NFULL + 1)
        do_unit(NFULL, NFULL * C, TAIL, lo, hi)


def kernel(items_emb, items, items_emb_grad):
    items = items.astype(jnp.int32)

    mesh = plsc.VectorSubcoreMesh(core_axis_name="c", subcore_axis_name="s")
    run = pl.kernel(
        _body,
        out_type=jax.ShapeDtypeStruct((DIM, M_ITEM), jnp.float32),
        mesh=mesh,
        scratch_types=[
            pltpu.VMEM((16, 16), jnp.int32),
            pltpu.VMEM((DIM, C), jnp.float32),
            pltpu.VMEM((B,), jnp.int32),
            pltpu.VMEM((2, B // 128, 8, 128), jnp.float32),
            pltpu.VMEM((256,), jnp.int32),
            pltpu.VMEM((HB,), jnp.int32),
            pltpu.VMEM((16, 256), jnp.int32),
            pltpu.VMEM_SHARED((16, 256), jnp.int32),
        ],
        compiler_params=pltpu.CompilerParams(
            use_tc_tiling_on_sc=False, needs_layout_passes=False),
    )
    grads4 = (items_emb_grad.T.reshape(2, 8, N_ROWS // 128, 128)
              .transpose(0, 2, 1, 3))
    out_t = run(items_emb.T, items, grads4)
    return out_t.T


# speedup vs baseline: 6.0562x; 6.0562x over previous
import jax
import jax.numpy as jnp
from jax import lax
from jax.experimental import pallas as pl
from jax.experimental.pallas import tpu as pltpu
from jax.experimental.pallas import tpu_sc as plsc

M_ITEM = 1_000_000
DIM = 16
N_ROWS = 819_200
LR = 0.01

LANES = 16
C = 4_096
CCOLS = C // 128
NFULL = 244
MAIN = 999_936
TCOLS = MAIN // 128
TAIL = M_ITEM - MAIN
GCOLS = N_ROWS // 128
NW = 32
B = 512
HS = N_ROWS // 16
HB = 2_048


def _bval(bounds2, u):
    lane = lax.broadcasted_iota(jnp.int32, (LANES,), 0)
    row = bounds2[u // LANES]
    return jnp.sum(jnp.where(lane == (u % LANES), row, 0))


def _body(emb4, items, grads4, tail_in, out4, out_tail,
          bounds2, chunk, kv, gblk, hist, hkv, comb, ttail, shist):
    c = lax.axis_index("c")
    sid = lax.axis_index("s")
    w = sid * 2 + c

    zeros16 = jnp.zeros((LANES,), jnp.int32)
    ones16 = jnp.ones((LANES,), jnp.int32)
    for i in range(16):
        hist[pl.ds(i * LANES, LANES)] = zeros16

    def hchunk(h, carry):
        base = pl.multiple_of(sid * HS + h * HB, 128)
        pltpu.sync_copy(items.at[pl.ds(base, HB)], hkv)
        for q in range(HB // LANES):
            keys = hkv[pl.ds(q * LANES, LANES)]
            u = lax.shift_right_logical(keys, 12)
            plsc.addupdate_scatter(hist, [u], ones16)
        return carry
    lax.fori_loop(0, HS // HB, hchunk, 0)

    pltpu.sync_copy(hist, shist.at[sid])
    plsc.subcore_barrier()
    pltpu.sync_copy(shist, comb)

    lane = lax.broadcasted_iota(jnp.int32, (LANES,), 0)
    carry0 = jnp.zeros((), jnp.int32)
    for i in range(16):
        seg = comb[0, pl.ds(i * LANES, LANES)]
        for t in range(1, 16):
            seg = seg + comb[t, pl.ds(i * LANES, LANES)]
        incl = plsc.cumsum(seg)
        bounds2[i] = (incl - seg) + carry0
        carry0 = carry0 + jnp.sum(jnp.where(lane == 15, incl, 0))

    def scan_blocks(i0, lo, hi, klo, khi, scatter16):
        lo_a = (lo // 128) * 128
        nblk = (hi - lo_a + (B - 1)) // B

        def blk(b, carry):
            s_nom = lo_a + b * B
            s = pl.multiple_of(jnp.minimum(s_nom, N_ROWS - B), 128)
            pltpu.sync_copy(items.at[pl.ds(s, B)], kv)
            pltpu.sync_copy(grads4.at[:, pl.ds(s // 128, B // 128)], gblk)
            glo = jnp.maximum(lo, s_nom)
            for q in range(B // LANES):
                keys = kv[pl.ds(q * LANES, LANES)]
                g = s + q * LANES + lax.broadcasted_iota(jnp.int32, (LANES,), 0)
                valid = ((g >= glo) & (g < hi)
                         & (keys >= klo) & (keys < khi))
                idx = jnp.where(valid, keys - i0, 0)
                scatter16(q, idx, valid)
            return carry
        lax.fori_loop(0, nblk, blk, 0)

    def do_unit(i0, tc0, lo, hi, klo, khi, wb_off, wb_tc0, wb_cols):
        pltpu.sync_copy(emb4.at[:, pl.ds(tc0, CCOLS)], chunk)

        def scat(q, idx, valid):
            col = lax.shift_right_logical(idx, 7)
            ln = idx & 127
            for d in range(DIM):
                v = gblk[d // 8, q // 8, d % 8,
                         pl.ds((q % 8) * LANES, LANES)] * (-LR)
                plsc.addupdate_scatter(
                    chunk,
                    [jnp.full((LANES,), d // 8, jnp.int32), col,
                     jnp.full((LANES,), d % 8, jnp.int32), ln],
                    v, mask=valid)
        scan_blocks(i0, lo, hi, klo, khi, scat)

        pltpu.sync_copy(chunk.at[:, pl.ds(wb_off, wb_cols)],
                        out4.at[:, pl.ds(wb_tc0, wb_cols)])

    my_units = (NFULL - w + (NW - 1)) // NW

    def unit_body(k, carry):
        u = w + k * NW
        i0 = pl.multiple_of(u * C, 128)
        tc0 = pl.multiple_of(u * CCOLS, 4)
        lo = _bval(bounds2, u)
        hi = _bval(bounds2, u + 1)
        do_unit(i0, tc0, lo, hi, i0, i0 + C, 0, tc0, CCOLS)
        return carry
    lax.fori_loop(0, my_units, unit_body, 0)

    @pl.when(w == NW - 1)
    def _():
        lo = _bval(bounds2, NFULL)
        i0 = MAIN - C
        do_unit(jnp.int32(i0), TCOLS - CCOLS, lo, N_ROWS,
                jnp.int32(NFULL * C), jnp.int32(MAIN),
                CCOLS - 4, TCOLS - 4, 4)

        pltpu.sync_copy(tail_in, ttail)

        def scat_tail(q, idx, valid):
            for d in range(DIM):
                v = gblk[d // 8, q // 8, d % 8,
                         pl.ds((q % 8) * LANES, LANES)] * (-LR)
                plsc.addupdate_scatter(
                    ttail, [jnp.full((LANES,), d, jnp.int32), idx],
                    v, mask=valid)
        scan_blocks(jnp.int32(MAIN), lo, N_ROWS,
                    jnp.int32(MAIN), jnp.int32(M_ITEM), scat_tail)
        pltpu.sync_copy(ttail, out_tail)


def kernel(items_emb, items, items_emb_grad):
    items = items.astype(jnp.int32)

    mesh = plsc.VectorSubcoreMesh(core_axis_name="c", subcore_axis_name="s")
    run = pl.kernel(
        _body,
        out_type=(jax.ShapeDtypeStruct((2, TCOLS, 8, 128), jnp.float32),
                  jax.ShapeDtypeStruct((DIM, TAIL), jnp.float32)),
        mesh=mesh,
        scratch_types=[
            pltpu.VMEM((16, 16), jnp.int32),
            pltpu.VMEM((2, CCOLS, 8, 128), jnp.float32),
            pltpu.VMEM((B,), jnp.int32),
            pltpu.VMEM((2, B // 128, 8, 128), jnp.float32),
            pltpu.VMEM((256,), jnp.int32),
            pltpu.VMEM((HB,), jnp.int32),
            pltpu.VMEM((16, 256), jnp.int32),
            pltpu.VMEM((DIM, TAIL), jnp.float32),
            pltpu.VMEM_SHARED((16, 256), jnp.int32),
        ],
        compiler_params=pltpu.CompilerParams(
            use_tc_tiling_on_sc=False, needs_layout_passes=False),
    )
    emb4 = (items_emb[:MAIN].T.reshape(2, 8, TCOLS, 128)
            .transpose(0, 2, 1, 3))
    grads4 = (items_emb_grad.T.reshape(2, 8, GCOLS, 128)
              .transpose(0, 2, 1, 3))
    tail_in = items_emb[MAIN:].T
    out4, out_tail = run(emb4, items, grads4, tail_in)
    main_t = out4.transpose(0, 2, 1, 3).reshape(DIM, MAIN)
    return jnp.concatenate([main_t, out_tail], axis=1).T


# speedup vs baseline: 6.3435x; 1.0474x over previous
import jax
import jax.numpy as jnp
from jax import lax
from jax.experimental import pallas as pl
from jax.experimental.pallas import tpu as pltpu
from jax.experimental.pallas import tpu_sc as plsc

M_ITEM = 1_000_000
DIM = 16
N_ROWS = 819_200
LR = 0.01

LANES = 16
C = 4_096
CCOLS = C // 128
NFULL = 244
MAIN = 999_936
TCOLS = MAIN // 128
TAIL = M_ITEM - MAIN
GCOLS = N_ROWS // 128
NW = 32
B = 512
HS = N_ROWS // 16
HB = 2_048


def _bval(bounds2, u):
    lane = lax.broadcasted_iota(jnp.int32, (LANES,), 0)
    row = bounds2[u // LANES]
    return jnp.sum(jnp.where(lane == (u % LANES), row, 0))


def _body(emb4, items, grads4, tail_in, out4, out_tail,
          bounds2, chunk, kv, gblk, hist, hkv, comb, ttail, shist,
          sem_k, sem_g):
    c = lax.axis_index("c")
    sid = lax.axis_index("s")
    w = sid * 2 + c

    zeros16 = jnp.zeros((LANES,), jnp.int32)
    ones16 = jnp.ones((LANES,), jnp.int32)
    for i in range(16):
        hist[pl.ds(i * LANES, LANES)] = zeros16

    def hchunk(h, carry):
        base = pl.multiple_of(sid * HS + h * HB, 128)
        pltpu.sync_copy(items.at[pl.ds(base, HB)], hkv)
        for q in range(HB // LANES):
            keys = hkv[pl.ds(q * LANES, LANES)]
            u = lax.shift_right_logical(keys, 12)
            plsc.addupdate_scatter(hist, [u], ones16)
        return carry
    lax.fori_loop(0, HS // HB, hchunk, 0)

    pltpu.sync_copy(hist, shist.at[sid])
    plsc.subcore_barrier()
    pltpu.sync_copy(shist, comb)

    lane = lax.broadcasted_iota(jnp.int32, (LANES,), 0)
    carry0 = jnp.zeros((), jnp.int32)
    for i in range(16):
        seg = comb[0, pl.ds(i * LANES, LANES)]
        for t in range(1, 16):
            seg = seg + comb[t, pl.ds(i * LANES, LANES)]
        incl = plsc.cumsum(seg)
        bounds2[i] = (incl - seg) + carry0
        carry0 = carry0 + jnp.sum(jnp.where(lane == 15, incl, 0))

    def scan_blocks(i0, lo, hi, klo, khi, scatter16):
        lo_a = (lo // 128) * 128
        nblk = (hi - lo_a + (B - 1)) // B

        def blk(b, carry):
            s_nom = lo_a + b * B
            s = pl.multiple_of(jnp.minimum(s_nom, N_ROWS - B), 128)
            d0 = pltpu.async_copy(items.at[pl.ds(s, B)], kv, sem_k)
            d1 = pltpu.async_copy(grads4.at[:, pl.ds(s // 128, B // 128)],
                                  gblk, sem_g)
            d0.wait()
            d1.wait()
            glo = jnp.maximum(lo, s_nom)
            for q in range(B // LANES):
                keys = kv[pl.ds(q * LANES, LANES)]
                g = s + q * LANES + lax.broadcasted_iota(jnp.int32, (LANES,), 0)
                valid = ((g >= glo) & (g < hi)
                         & (keys >= klo) & (keys < khi))
                idx = jnp.where(valid, keys - i0, 0)
                scatter16(q, idx, valid)
            return carry
        lax.fori_loop(0, nblk, blk, 0)

    def do_unit(i0, tc0, lo, hi, klo, khi, wb_off, wb_tc0, wb_cols):
        pltpu.sync_copy(emb4.at[:, pl.ds(tc0, CCOLS)], chunk)

        def scat(q, idx, valid):
            col = lax.shift_right_logical(idx, 7)
            ln = idx & 127
            for d in range(DIM):
                v = gblk[d // 8, q // 8, d % 8,
                         pl.ds((q % 8) * LANES, LANES)] * (-LR)
                plsc.addupdate_scatter(
                    chunk,
                    [jnp.full((LANES,), d // 8, jnp.int32), col,
                     jnp.full((LANES,), d % 8, jnp.int32), ln],
                    v, mask=valid)
        scan_blocks(i0, lo, hi, klo, khi, scat)

        pltpu.sync_copy(chunk.at[:, pl.ds(wb_off, wb_cols)],
                        out4.at[:, pl.ds(wb_tc0, wb_cols)])

    my_units = (NFULL - w + (NW - 1)) // NW

    def unit_body(k, carry):
        u = w + k * NW
        i0 = pl.multiple_of(u * C, 128)
        tc0 = pl.multiple_of(u * CCOLS, 4)
        lo = _bval(bounds2, u)
        hi = _bval(bounds2, u + 1)
        do_unit(i0, tc0, lo, hi, i0, i0 + C, 0, tc0, CCOLS)
        return carry
    lax.fori_loop(0, my_units, unit_body, 0)

    @pl.when(w == NW - 1)
    def _():
        lo = _bval(bounds2, NFULL)
        i0 = MAIN - C
        do_unit(jnp.int32(i0), TCOLS - CCOLS, lo, N_ROWS,
                jnp.int32(NFULL * C), jnp.int32(MAIN),
                CCOLS - 4, TCOLS - 4, 4)

        pltpu.sync_copy(tail_in, ttail)

        def scat_tail(q, idx, valid):
            for d in range(DIM):
                v = gblk[d // 8, q // 8, d % 8,
                         pl.ds((q % 8) * LANES, LANES)] * (-LR)
                plsc.addupdate_scatter(
                    ttail, [jnp.full((LANES,), d, jnp.int32), idx],
                    v, mask=valid)
        scan_blocks(jnp.int32(MAIN), lo, N_ROWS,
                    jnp.int32(MAIN), jnp.int32(M_ITEM), scat_tail)
        pltpu.sync_copy(ttail, out_tail)


def kernel(items_emb, items, items_emb_grad):
    items = items.astype(jnp.int32)

    mesh = plsc.VectorSubcoreMesh(core_axis_name="c", subcore_axis_name="s")
    run = pl.kernel(
        _body,
        out_type=(jax.ShapeDtypeStruct((2, TCOLS, 8, 128), jnp.float32),
                  jax.ShapeDtypeStruct((DIM, TAIL), jnp.float32)),
        mesh=mesh,
        scratch_types=[
            pltpu.VMEM((16, 16), jnp.int32),
            pltpu.VMEM((2, CCOLS, 8, 128), jnp.float32),
            pltpu.VMEM((B,), jnp.int32),
            pltpu.VMEM((2, B // 128, 8, 128), jnp.float32),
            pltpu.VMEM((256,), jnp.int32),
            pltpu.VMEM((HB,), jnp.int32),
            pltpu.VMEM((16, 256), jnp.int32),
            pltpu.VMEM((DIM, TAIL), jnp.float32),
            pltpu.VMEM_SHARED((16, 256), jnp.int32),
            pltpu.SemaphoreType.DMA,
            pltpu.SemaphoreType.DMA,
        ],
        compiler_params=pltpu.CompilerParams(
            use_tc_tiling_on_sc=False, needs_layout_passes=False),
    )
    emb4 = (items_emb[:MAIN].T.reshape(2, 8, TCOLS, 128)
            .transpose(0, 2, 1, 3))
    grads4 = (items_emb_grad.T.reshape(2, 8, GCOLS, 128)
              .transpose(0, 2, 1, 3))
    tail_in = items_emb[MAIN:].T
    out4, out_tail = run(emb4, items, grads4, tail_in)
    main_t = out4.transpose(0, 2, 1, 3).reshape(DIM, MAIN)
    return jnp.concatenate([main_t, out_tail], axis=1).T


# speedup vs baseline: 6.8130x; 1.0740x over previous
import jax
import jax.numpy as jnp
from jax import lax
from jax.experimental import pallas as pl
from jax.experimental.pallas import tpu as pltpu
from jax.experimental.pallas import tpu_sc as plsc

M_ITEM = 1_000_000
DIM = 16
N_ROWS = 819_200
LR = 0.01

LANES = 16
C = 4_096
CCOLS = C // 128
NFULL = 244
MAIN = 999_936
TCOLS = MAIN // 128
TAIL = M_ITEM - MAIN
GCOLS = N_ROWS // 128
NW = 32
B = 512
HS = N_ROWS // 16
HB = 2_048


def _bval(bounds2, u):
    lane = lax.broadcasted_iota(jnp.int32, (LANES,), 0)
    row = bounds2[u // LANES]
    return jnp.sum(jnp.where(lane == (u % LANES), row, 0))


def _body(emb4, items, grads4, tail_in, out4, out_tail,
          bounds2, chunk, kv, gblk, hist, hkv, comb, ttail, shist,
          sem_k0, sem_g0, sem_k1, sem_g1):
    c = lax.axis_index("c")
    sid = lax.axis_index("s")
    w = sid * 2 + c

    zeros16 = jnp.zeros((LANES,), jnp.int32)
    ones16 = jnp.ones((LANES,), jnp.int32)
    for i in range(16):
        hist[pl.ds(i * LANES, LANES)] = zeros16

    def hchunk(h, carry):
        base = pl.multiple_of(sid * HS + h * HB, 128)
        pltpu.sync_copy(items.at[pl.ds(base, HB)], hkv)
        for q in range(HB // LANES):
            keys = hkv[pl.ds(q * LANES, LANES)]
            u = lax.shift_right_logical(keys, 12)
            plsc.addupdate_scatter(hist, [u], ones16)
        return carry
    lax.fori_loop(0, HS // HB, hchunk, 0)

    pltpu.sync_copy(hist, shist.at[sid])
    plsc.subcore_barrier()
    pltpu.sync_copy(shist, comb)

    lane = lax.broadcasted_iota(jnp.int32, (LANES,), 0)
    carry0 = jnp.zeros((), jnp.int32)
    for i in range(16):
        seg = comb[0, pl.ds(i * LANES, LANES)]
        for t in range(1, 16):
            seg = seg + comb[t, pl.ds(i * LANES, LANES)]
        incl = plsc.cumsum(seg)
        bounds2[i] = (incl - seg) + carry0
        carry0 = carry0 + jnp.sum(jnp.where(lane == 15, incl, 0))

    sems = ((sem_k0, sem_g0), (sem_k1, sem_g1))

    def _issue(slot, s):
        pltpu.async_copy(items.at[pl.ds(s, B)], kv.at[slot], sems[slot][0])
        pltpu.async_copy(grads4.at[:, pl.ds(s // 128, B // 128)],
                         gblk.at[slot], sems[slot][1])

    def _drain(slot, s):
        pltpu.make_async_copy(items.at[pl.ds(s, B)], kv.at[slot],
                              sems[slot][0]).wait()
        pltpu.make_async_copy(grads4.at[:, pl.ds(s // 128, B // 128)],
                              gblk.at[slot], sems[slot][1]).wait()

    def scan_blocks(i0, lo, hi, klo, khi, scatter16):
        lo_a = (lo // 128) * 128
        nblk = (hi - lo_a + (B - 1)) // B

        def s_of(b):
            return pl.multiple_of(
                jnp.minimum(lo_a + b * B, N_ROWS - B), 128)

        @pl.when(nblk > 0)
        def _():
            _issue(0, s_of(0))

        def blk(b, carry):
            p = b % 2
            s_nom = lo_a + b * B
            s = s_of(b)

            @pl.when(b + 1 < nblk)
            def _():
                s_n = s_of(b + 1)

                @pl.when(p == 0)
                def _():
                    _issue(1, s_n)

                @pl.when(p == 1)
                def _():
                    _issue(0, s_n)

            @pl.when(p == 0)
            def _():
                _drain(0, s)

            @pl.when(p == 1)
            def _():
                _drain(1, s)

            glo = jnp.maximum(lo, s_nom)
            for q in range(B // LANES):
                keys = kv[p, pl.ds(q * LANES, LANES)]
                g = s + q * LANES + lax.broadcasted_iota(jnp.int32, (LANES,), 0)
                valid = ((g >= glo) & (g < hi)
                         & (keys >= klo) & (keys < khi))
                idx = jnp.where(valid, keys - i0, 0)
                scatter16(p, q, idx, valid)
            return carry
        lax.fori_loop(0, nblk, blk, 0)

    def do_unit(i0, tc0, lo, hi, klo, khi, wb_off, wb_tc0, wb_cols):
        pltpu.sync_copy(emb4.at[:, pl.ds(tc0, CCOLS)], chunk)

        def scat(p, q, idx, valid):
            col = lax.shift_right_logical(idx, 7)
            ln = idx & 127
            for d in range(DIM):
                v = gblk[p, d // 8, q // 8, d % 8,
                         pl.ds((q % 8) * LANES, LANES)] * (-LR)
                plsc.addupdate_scatter(
                    chunk,
                    [jnp.full((LANES,), d // 8, jnp.int32), col,
                     jnp.full((LANES,), d % 8, jnp.int32), ln],
                    v, mask=valid)
        scan_blocks(i0, lo, hi, klo, khi, scat)

        pltpu.sync_copy(chunk.at[:, pl.ds(wb_off, wb_cols)],
                        out4.at[:, pl.ds(wb_tc0, wb_cols)])

    my_units = (NFULL - w + (NW - 1)) // NW

    def unit_body(k, carry):
        u = w + k * NW
        i0 = pl.multiple_of(u * C, 128)
        tc0 = pl.multiple_of(u * CCOLS, 4)
        lo = _bval(bounds2, u)
        hi = _bval(bounds2, u + 1)
        do_unit(i0, tc0, lo, hi, i0, i0 + C, 0, tc0, CCOLS)
        return carry
    lax.fori_loop(0, my_units, unit_body, 0)

    @pl.when(w == NW - 1)
    def _():
        lo = _bval(bounds2, NFULL)
        i0 = MAIN - C
        do_unit(jnp.int32(i0), TCOLS - CCOLS, lo, N_ROWS,
                jnp.int32(NFULL * C), jnp.int32(MAIN),
                CCOLS - 4, TCOLS - 4, 4)

        pltpu.sync_copy(tail_in, ttail)

        def scat_tail(p, q, idx, valid):
            for d in range(DIM):
                v = gblk[p, d // 8, q // 8, d % 8,
                         pl.ds((q % 8) * LANES, LANES)] * (-LR)
                plsc.addupdate_scatter(
                    ttail, [jnp.full((LANES,), d, jnp.int32), idx],
                    v, mask=valid)
        scan_blocks(jnp.int32(MAIN), lo, N_ROWS,
                    jnp.int32(MAIN), jnp.int32(M_ITEM), scat_tail)
        pltpu.sync_copy(ttail, out_tail)


def kernel(items_emb, items, items_emb_grad):
    items = items.astype(jnp.int32)

    mesh = plsc.VectorSubcoreMesh(core_axis_name="c", subcore_axis_name="s")
    run = pl.kernel(
        _body,
        out_type=(jax.ShapeDtypeStruct((2, TCOLS, 8, 128), jnp.float32),
                  jax.ShapeDtypeStruct((DIM, TAIL), jnp.float32)),
        mesh=mesh,
        scratch_types=[
            pltpu.VMEM((16, 16), jnp.int32),
            pltpu.VMEM((2, CCOLS, 8, 128), jnp.float32),
            pltpu.VMEM((2, B), jnp.int32),
            pltpu.VMEM((2, 2, B // 128, 8, 128), jnp.float32),
            pltpu.VMEM((256,), jnp.int32),
            pltpu.VMEM((HB,), jnp.int32),
            pltpu.VMEM((16, 256), jnp.int32),
            pltpu.VMEM((DIM, TAIL), jnp.float32),
            pltpu.VMEM_SHARED((16, 256), jnp.int32),
            pltpu.SemaphoreType.DMA,
            pltpu.SemaphoreType.DMA,
            pltpu.SemaphoreType.DMA,
            pltpu.SemaphoreType.DMA,
        ],
        compiler_params=pltpu.CompilerParams(
            use_tc_tiling_on_sc=False, needs_layout_passes=False),
    )
    emb4 = (items_emb[:MAIN].T.reshape(2, 8, TCOLS, 128)
            .transpose(0, 2, 1, 3))
    grads4 = (items_emb_grad.T.reshape(2, 8, GCOLS, 128)
              .transpose(0, 2, 1, 3))
    tail_in = items_emb[MAIN:].T
    out4, out_tail = run(emb4, items, grads4, tail_in)
    main_t = out4.transpose(0, 2, 1, 3).reshape(DIM, MAIN)
    return jnp.concatenate([main_t, out_tail], axis=1).T


# speedup vs baseline: 7.0460x; 1.0342x over previous
import jax
import jax.numpy as jnp
from jax import lax
from jax.experimental import pallas as pl
from jax.experimental.pallas import tpu as pltpu
from jax.experimental.pallas import tpu_sc as plsc

M_ITEM = 1_000_000
DIM = 16
N_ROWS = 819_200
LR = 0.01

LANES = 16
C = 4_096
CCOLS = C // 128
NFULL = 244
MAIN = 999_936
TCOLS = MAIN // 128
TAIL = M_ITEM - MAIN
GCOLS = N_ROWS // 128
NW = 32
B = 512
HS = N_ROWS // 16
HB = 2_048


def _bval(bounds2, u):
    lane = lax.broadcasted_iota(jnp.int32, (LANES,), 0)
    row = bounds2[u // LANES]
    return jnp.sum(jnp.where(lane == (u % LANES), row, 0))


def _body(emb4, items, grads4, tail_in, out4, out_tail,
          bounds2, chunk, kv, gblk, hist, hkv, comb, ttail, shist,
          sem_k0, sem_g0, sem_k1, sem_g1):
    c = lax.axis_index("c")
    sid = lax.axis_index("s")
    w = sid * 2 + c

    zeros16 = jnp.zeros((LANES,), jnp.int32)
    ones16 = jnp.ones((LANES,), jnp.int32)
    for i in range(16):
        hist[pl.ds(i * LANES, LANES)] = zeros16

    hsems = (sem_k0, sem_k1)

    def _hissue(slot, h):
        base = pl.multiple_of(sid * HS + h * HB, 128)
        pltpu.async_copy(items.at[pl.ds(base, HB)], hkv.at[slot], hsems[slot])

    def _hdrain(slot, h):
        base = pl.multiple_of(sid * HS + h * HB, 128)
        pltpu.make_async_copy(items.at[pl.ds(base, HB)], hkv.at[slot],
                              hsems[slot]).wait()

    NH = HS // HB
    _hissue(0, 0)

    def hchunk(h, carry):
        p = h % 2

        @pl.when(h + 1 < NH)
        def _():
            @pl.when(p == 0)
            def _():
                _hissue(1, h + 1)

            @pl.when(p == 1)
            def _():
                _hissue(0, h + 1)

        @pl.when(p == 0)
        def _():
            _hdrain(0, h)

        @pl.when(p == 1)
        def _():
            _hdrain(1, h)

        for q in range(HB // LANES):
            keys = hkv[p, pl.ds(q * LANES, LANES)]
            u = lax.shift_right_logical(keys, 12)
            plsc.addupdate_scatter(hist, [u], ones16)
        return carry
    lax.fori_loop(0, NH, hchunk, 0)

    pltpu.sync_copy(hist, shist.at[sid])
    plsc.subcore_barrier()
    pltpu.sync_copy(shist, comb)

    lane = lax.broadcasted_iota(jnp.int32, (LANES,), 0)
    carry0 = jnp.zeros((), jnp.int32)
    for i in range(16):
        seg = comb[0, pl.ds(i * LANES, LANES)]
        for t in range(1, 16):
            seg = seg + comb[t, pl.ds(i * LANES, LANES)]
        incl = plsc.cumsum(seg)
        bounds2[i] = (incl - seg) + carry0
        carry0 = carry0 + jnp.sum(jnp.where(lane == 15, incl, 0))

    sems = ((sem_k0, sem_g0), (sem_k1, sem_g1))

    def _issue(slot, s):
        pltpu.async_copy(items.at[pl.ds(s, B)], kv.at[slot], sems[slot][0])
        pltpu.async_copy(grads4.at[:, pl.ds(s // 128, B // 128)],
                         gblk.at[slot], sems[slot][1])

    def _drain(slot, s):
        pltpu.make_async_copy(items.at[pl.ds(s, B)], kv.at[slot],
                              sems[slot][0]).wait()
        pltpu.make_async_copy(grads4.at[:, pl.ds(s // 128, B // 128)],
                              gblk.at[slot], sems[slot][1]).wait()

    def scan_blocks(i0, lo, hi, klo, khi, scatter16):
        lo_a = (lo // 128) * 128
        nblk = (hi - lo_a + (B - 1)) // B

        def s_of(b):
            return pl.multiple_of(
                jnp.minimum(lo_a + b * B, N_ROWS - B), 128)

        @pl.when(nblk > 0)
        def _():
            _issue(0, s_of(0))

        def blk(b, carry):
            p = b % 2
            s_nom = lo_a + b * B
            s = s_of(b)

            @pl.when(b + 1 < nblk)
            def _():
                s_n = s_of(b + 1)

                @pl.when(p == 0)
                def _():
                    _issue(1, s_n)

                @pl.when(p == 1)
                def _():
                    _issue(0, s_n)

            @pl.when(p == 0)
            def _():
                _drain(0, s)

            @pl.when(p == 1)
            def _():
                _drain(1, s)

            glo = jnp.maximum(lo, s_nom)
            for q in range(B // LANES):
                keys = kv[p, pl.ds(q * LANES, LANES)]
                g = s + q * LANES + lax.broadcasted_iota(jnp.int32, (LANES,), 0)
                valid = ((g >= glo) & (g < hi)
                         & (keys >= klo) & (keys < khi))
                idx = jnp.where(valid, keys - i0, 0)
                scatter16(p, q, idx, valid)
            return carry
        lax.fori_loop(0, nblk, blk, 0)

    def do_unit(i0, tc0, lo, hi, klo, khi, wb_off, wb_tc0, wb_cols):
        pltpu.sync_copy(emb4.at[:, pl.ds(tc0, CCOLS)], chunk)

        def scat(p, q, idx, valid):
            col = lax.shift_right_logical(idx, 7)
            ln = idx & 127
            for d in range(DIM):
                v = gblk[p, d // 8, q // 8, d % 8,
                         pl.ds((q % 8) * LANES, LANES)] * (-LR)
                plsc.addupdate_scatter(
                    chunk,
                    [jnp.full((LANES,), d // 8, jnp.int32), col,
                     jnp.full((LANES,), d % 8, jnp.int32), ln],
                    v, mask=valid)
        scan_blocks(i0, lo, hi, klo, khi, scat)

        pltpu.sync_copy(chunk.at[:, pl.ds(wb_off, wb_cols)],
                        out4.at[:, pl.ds(wb_tc0, wb_cols)])

    my_units = (NFULL - w + (NW - 1)) // NW

    def unit_body(k, carry):
        u = w + k * NW
        i0 = pl.multiple_of(u * C, 128)
        tc0 = pl.multiple_of(u * CCOLS, 4)
        lo = _bval(bounds2, u)
        hi = _bval(bounds2, u + 1)
        do_unit(i0, tc0, lo, hi, i0, i0 + C, 0, tc0, CCOLS)
        return carry
    lax.fori_loop(0, my_units, unit_body, 0)

    @pl.when(w == NW - 1)
    def _():
        lo = _bval(bounds2, NFULL)
        i0 = MAIN - C
        do_unit(jnp.int32(i0), TCOLS - CCOLS, lo, N_ROWS,
                jnp.int32(NFULL * C), jnp.int32(MAIN),
                CCOLS - 4, TCOLS - 4, 4)

        pltpu.sync_copy(tail_in, ttail)

        def scat_tail(p, q, idx, valid):
            for d in range(DIM):
                v = gblk[p, d // 8, q // 8, d % 8,
                         pl.ds((q % 8) * LANES, LANES)] * (-LR)
                plsc.addupdate_scatter(
                    ttail, [jnp.full((LANES,), d, jnp.int32), idx],
                    v, mask=valid)
        scan_blocks(jnp.int32(MAIN), lo, N_ROWS,
                    jnp.int32(MAIN), jnp.int32(M_ITEM), scat_tail)
        pltpu.sync_copy(ttail, out_tail)


def kernel(items_emb, items, items_emb_grad):
    items = items.astype(jnp.int32)

    mesh = plsc.VectorSubcoreMesh(core_axis_name="c", subcore_axis_name="s")
    run = pl.kernel(
        _body,
        out_type=(jax.ShapeDtypeStruct((2, TCOLS, 8, 128), jnp.float32),
                  jax.ShapeDtypeStruct((DIM, TAIL), jnp.float32)),
        mesh=mesh,
        scratch_types=[
            pltpu.VMEM((16, 16), jnp.int32),
            pltpu.VMEM((2, CCOLS, 8, 128), jnp.float32),
            pltpu.VMEM((2, B), jnp.int32),
            pltpu.VMEM((2, 2, B // 128, 8, 128), jnp.float32),
            pltpu.VMEM((256,), jnp.int32),
            pltpu.VMEM((2, HB), jnp.int32),
            pltpu.VMEM((16, 256), jnp.int32),
            pltpu.VMEM((DIM, TAIL), jnp.float32),
            pltpu.VMEM_SHARED((16, 256), jnp.int32),
            pltpu.SemaphoreType.DMA,
            pltpu.SemaphoreType.DMA,
            pltpu.SemaphoreType.DMA,
            pltpu.SemaphoreType.DMA,
        ],
        compiler_params=pltpu.CompilerParams(
            use_tc_tiling_on_sc=False, needs_layout_passes=False),
    )
    emb4 = (items_emb[:MAIN].T.reshape(2, 8, TCOLS, 128)
            .transpose(0, 2, 1, 3))
    grads4 = (items_emb_grad.T.reshape(2, 8, GCOLS, 128)
              .transpose(0, 2, 1, 3))
    tail_in = items_emb[MAIN:].T
    out4, out_tail = run(emb4, items, grads4, tail_in)
    main_t = out4.transpose(0, 2, 1, 3).reshape(DIM, MAIN)
    return jnp.concatenate([main_t, out_tail], axis=1).T
